# Initial kernel scaffold; baseline (speedup 1.0000x reference)
import jax
import jax.numpy as jnp
from jax.experimental import pallas as pl


def kernel(x, edge_index0, edge_type0, edge_index1, edge_type1, et_table, ea_table, g0, b0, Wq0, Wk0, Wv0, Wea0, Wet0, Wo0, Ws0, Wq1, Wk1, Wv1, Wea1, Wet1, Wo1, Ws1):
    raise NotImplementedError("write your pallas kernel here")


# speedup vs baseline: 17.7678x; 17.7678x over previous
import functools
import math

import jax
import jax.numpy as jnp
from jax import lax
from jax.experimental import pallas as pl
from jax.experimental.pallas import tpu as pltpu
from jax.experimental.pallas import tpu_sc as plsc

N0, N1, N2, D, H = 10000, 5000, 1000, 128, 4
DH = D // H
E0, E1 = 320000, 80000

NC, NS = 2, 16
NW = NC * NS
CH = 128
ACC_W = 256

f32 = jnp.float32
i32 = jnp.int32


def _ceil_to(x, m):
    return ((x + m - 1) // m) * m


def _mmcat(a, w1, w2):
    n = a.shape[0]
    bn = 200
    assert n % bn == 0

    def body(a_ref, w1_ref, w2_ref, o_ref):
        av = a_ref[...]
        o_ref[:, :D] = jnp.dot(av, w1_ref[...], preferred_element_type=f32)
        o_ref[:, D:] = jnp.dot(av, w2_ref[...], preferred_element_type=f32)

    return pl.pallas_call(
        body,
        grid=(n // bn,),
        in_specs=[
            pl.BlockSpec((bn, D), lambda i: (i, 0)),
            pl.BlockSpec((D, D), lambda i: (0, 0)),
            pl.BlockSpec((D, D), lambda i: (0, 0)),
        ],
        out_specs=pl.BlockSpec((bn, 2 * D), lambda i: (i, 0)),
        out_shape=jax.ShapeDtypeStruct((n, 2 * D), f32),
    )(a, w1, w2)


def _mm2(a, w1, w2):
    n = a.shape[0]
    bn = 200
    assert n % bn == 0

    def body(a_ref, w1_ref, w2_ref, o1_ref, o2_ref):
        av = a_ref[...]
        o1_ref[...] = jnp.dot(av, w1_ref[...], preferred_element_type=f32)
        o2_ref[...] = jnp.dot(av, w2_ref[...], preferred_element_type=f32)

    return pl.pallas_call(
        body,
        grid=(n // bn,),
        in_specs=[
            pl.BlockSpec((bn, D), lambda i: (i, 0)),
            pl.BlockSpec((D, D), lambda i: (0, 0)),
            pl.BlockSpec((D, D), lambda i: (0, 0)),
        ],
        out_specs=[
            pl.BlockSpec((bn, D), lambda i: (i, 0)),
            pl.BlockSpec((bn, D), lambda i: (i, 0)),
        ],
        out_shape=[
            jax.ShapeDtypeStruct((n, D), f32),
            jax.ShapeDtypeStruct((n, D), f32),
        ],
    )(a, w1, w2)


def _edge_math(kve, qe, t3, ea_table, et_table, wea, wet, e_real):
    ep = kve.shape[0]
    be = 512
    assert ep % be == 0
    scale = 1.0 / math.sqrt(DH)

    def body(kve_ref, qe_ref, t_ref, ea_ref, et_ref, wea_ref, wet_ref,
             wv_ref, ww_ref):
        pid = pl.program_id(0)
        tk = jnp.dot(ea_ref[...], wea_ref[...], preferred_element_type=f32)
        tv = jnp.dot(et_ref[...], wet_ref[...], preferred_element_type=f32)
        t = t_ref[0, 0, :]
        oh = (t[:, None] == lax.broadcasted_iota(i32, (be, 4), 1)).astype(f32)
        ke = kve_ref[:, :D] + jnp.dot(oh, tk, preferred_element_type=f32)
        ve = kve_ref[:, D:] + jnp.dot(oh, tv, preferred_element_type=f32)
        hsel = (lax.broadcasted_iota(i32, (D, H), 0) // DH
                == lax.broadcasted_iota(i32, (D, H), 1)).astype(f32)
        logits = jnp.dot(qe_ref[...] * ke, hsel,
                         preferred_element_type=f32) * scale
        w = jnp.exp(logits)
        eid = pid * be + lax.broadcasted_iota(i32, (be, 1), 0)
        w = jnp.where(eid < e_real, w, 0.0)
        wexp = jnp.dot(w, hsel.T, preferred_element_type=f32)
        wv_ref[...] = wexp * ve
        ww_ref[...] = wexp

    return pl.pallas_call(
        body,
        grid=(ep // be,),
        in_specs=[
            pl.BlockSpec((be, 2 * D), lambda i: (i, 0)),
            pl.BlockSpec((be, D), lambda i: (i, 0)),
            pl.BlockSpec((1, 1, be), lambda i: (i, 0, 0)),
            pl.BlockSpec((4, D), lambda i: (0, 0)),
            pl.BlockSpec((4, D), lambda i: (0, 0)),
            pl.BlockSpec((D, D), lambda i: (0, 0)),
            pl.BlockSpec((D, D), lambda i: (0, 0)),
        ],
        out_specs=[
            pl.BlockSpec((be, D), lambda i: (i, 0)),
            pl.BlockSpec((be, D), lambda i: (i, 0)),
        ],
        out_shape=[
            jax.ShapeDtypeStruct((ep, D), f32),
            jax.ShapeDtypeStruct((ep, D), f32),
        ],
    )(kve, qe, t3, ea_table, et_table, wea, wet)


def _epilogue(accs, wo, skip, g, b, n_dst, n_dst_p, with_bn):
    accv, accw = accs

    def body(accv_ref, accw_ref, wo_ref, skip_ref, g_ref, b_ref, o_ref):
        agg = accv_ref[pl.ds(0, n_dst), :] + accv_ref[pl.ds(n_dst_p, n_dst), :]
        den = (accw_ref[pl.ds(0, n_dst), :]
               + accw_ref[pl.ds(n_dst_p, n_dst), :] + 1e-16)
        h = jnp.dot(agg / den, wo_ref[...], preferred_element_type=f32)
        h = h + skip_ref[...]
        if with_bn:
            mu = jnp.mean(h, axis=0, keepdims=True)
            var = jnp.mean((h - mu) * (h - mu), axis=0, keepdims=True)
            h = (h - mu) * lax.rsqrt(var + 1e-5) * g_ref[...] + b_ref[...]
        o_ref[...] = h

    return pl.pallas_call(
        body,
        out_shape=jax.ShapeDtypeStruct((n_dst, D), f32),
    )(accv, accw, wo, skip, g, b)


@functools.cache
def _make_gather(ep, n_chunks):
    mesh = plsc.VectorSubcoreMesh(core_axis_name="c", subcore_axis_name="s")

    @functools.partial(
        pl.kernel,
        out_type=[
            jax.ShapeDtypeStruct((ep, 2 * D), f32),
            jax.ShapeDtypeStruct((ep, D), f32),
        ],
        mesh=mesh,
        scratch_types=[
            pltpu.VMEM((CH,), i32),
            pltpu.VMEM((CH,), i32),
            pltpu.VMEM((CH, 2 * D), f32),
            pltpu.VMEM((CH, D), f32),
            pltpu.SemaphoreType.DMA,
            pltpu.SemaphoreType.DMA,
        ],
    )
    def k(src1d, dst1d, kv_hbm, q_hbm, kve_out, qe_out,
          sidx, didx, kvbuf, qbuf, sem1, sem2):
        wid = lax.axis_index("s") * NC + lax.axis_index("c")
        rbase = wid * n_chunks

        def body(j, carry):
            ebase = (rbase + j) * CH
            pltpu.sync_copy(src1d.at[pl.ds(ebase, CH)], sidx)
            pltpu.sync_copy(dst1d.at[pl.ds(ebase, CH)], didx)
            c1 = pltpu.async_copy(kv_hbm.at[sidx], kvbuf, sem1)
            c2 = pltpu.async_copy(q_hbm.at[didx], qbuf, sem2)
            c1.wait()
            c2.wait()
            pltpu.sync_copy(kvbuf, kve_out.at[pl.ds(ebase, CH)])
            pltpu.sync_copy(qbuf, qe_out.at[pl.ds(ebase, CH)])
            return carry

        lax.fori_loop(0, n_chunks, body, 0)

    return k


@functools.cache
def _make_scatter(ep, n_chunks, n_dst_p):
    mesh = plsc.VectorSubcoreMesh(core_axis_name="c", subcore_axis_name="s")
    rpt = n_dst_p // NS
    dr = 64
    assert rpt % dr == 0

    @functools.partial(
        pl.kernel,
        out_type=jax.ShapeDtypeStruct((NC * n_dst_p, D), f32),
        mesh=mesh,
        scratch_types=[
            pltpu.VMEM((CH,), i32),
            pltpu.VMEM((CH, D), f32),
            pltpu.VMEM((dr, D), f32),
            pltpu.VMEM_SHARED((n_dst_p, D), f32),
            pltpu.SemaphoreType.DMA,
        ],
    )
    def k(dst1d, wv_hbm, acc_out, drow, buf, zbuf, shared, sem):
        cid = lax.axis_index("c")
        sid = lax.axis_index("s")
        wid = sid * NC + cid
        rbase = wid * n_chunks

        def zbody(r, carry):
            for c in range(D // 16):
                zbuf[r, pl.ds(c * 16, 16)] = jnp.zeros((16,), f32)
            return carry

        lax.fori_loop(0, dr, zbody, 0)
        for z in range(rpt // dr):
            pltpu.sync_copy(zbuf, shared.at[pl.ds(sid * rpt + z * dr, dr)])
        plsc.subcore_barrier()

        def body(j, carry):
            ebase = (rbase + j) * CH
            pltpu.sync_copy(wv_hbm.at[pl.ds(ebase, CH)], buf)
            pltpu.sync_copy(dst1d.at[pl.ds(ebase, CH)], drow)
            pltpu.sync_copy(buf, shared.at[drow], add=True)
            return carry

        lax.fori_loop(0, n_chunks, body, 0)
        plsc.subcore_barrier()

        for z in range(rpt // dr):
            pltpu.sync_copy(shared.at[pl.ds(sid * rpt + z * dr, dr)], zbuf)
            pltpu.sync_copy(
                zbuf, acc_out.at[pl.ds(cid * n_dst_p + sid * rpt + z * dr, dr)])

    return k


def _pad_edges(src, dst, typ, ep):
    e = src.shape[0]
    pad = ep - e
    src = jnp.concatenate([src, jnp.zeros((pad,), i32)])
    dst = jnp.concatenate([dst, jnp.zeros((pad,), i32)])
    typ = jnp.concatenate([typ, jnp.zeros((pad,), i32)])
    return src, dst, typ.reshape(ep // 512, 1, 512)


def _layer(x_src, x_dst, src2d, dst2d, t3, e_real, ep, n_dst, n_dst_p,
           ea_table, et_table, wq, wk, wv, wea, wet, wo, ws, g, b, with_bn):
    n_chunks = ep // (NW * CH)
    kv = _mmcat(x_src, wk, wv)
    q, skip = _mm2(x_dst, wq, ws)
    kve, qe = _make_gather(ep, n_chunks)(src2d, dst2d, kv, q)
    wvz, wwz = _edge_math(kve, qe, t3, ea_table, et_table, wea, wet, e_real)
    scat = _make_scatter(ep, n_chunks, n_dst_p)
    accv = scat(dst2d, wvz)
    accw = scat(dst2d, wwz)
    return _epilogue((accv, accw), wo, skip, g, b, n_dst, n_dst_p, with_bn)


def kernel(x, edge_index0, edge_type0, edge_index1, edge_type1,
           et_table, ea_table, g0, b0,
           Wq0, Wk0, Wv0, Wea0, Wet0, Wo0, Ws0,
           Wq1, Wk1, Wv1, Wea1, Wet1, Wo1, Ws1):
    ep0 = _ceil_to(E0, NW * CH * 4)
    ep1 = _ceil_to(E1, NW * CH * 4)
    n1p = _ceil_to(N1, NS * 8)
    n2p = _ceil_to(N2, NS * 8)
    g2 = g0.reshape(1, D)
    b2 = b0.reshape(1, D)

    s0, d0, t0 = _pad_edges(edge_index0[0], edge_index0[1], edge_type0, ep0)
    s1, d1, t1 = _pad_edges(edge_index1[0], edge_index1[1], edge_type1, ep1)

    h1 = _layer(x, x[:N1], s0, d0, t0, E0, ep0, N1, n1p,
                ea_table, et_table, Wq0, Wk0, Wv0, Wea0, Wet0, Wo0, Ws0,
                g2, b2, True)
    out = _layer(h1, h1[:N2], s1, d1, t1, E1, ep1, N2, n2p,
                 ea_table, et_table, Wq1, Wk1, Wv1, Wea1, Wet1, Wo1, Ws1,
                 g2, b2, False)
    return out


# speedup vs baseline: 21.7311x; 1.2231x over previous
import functools
import math

import jax
import jax.numpy as jnp
from jax import lax
from jax.experimental import pallas as pl
from jax.experimental.pallas import tpu as pltpu
from jax.experimental.pallas import tpu_sc as plsc

N0, N1, N2, D, H = 10000, 5000, 1000, 128, 4
DH = D // H
E0, E1 = 320000, 80000

NC, NS = 2, 16
NW = NC * NS
CH = 128
ACC_W = 256

f32 = jnp.float32
i32 = jnp.int32


def _ceil_to(x, m):
    return ((x + m - 1) // m) * m


def _mmcat(a, w1, w2):
    n = a.shape[0]
    bn = 200
    assert n % bn == 0

    def body(a_ref, w1_ref, w2_ref, o_ref):
        av = a_ref[...]
        o_ref[:, :D] = jnp.dot(av, w1_ref[...], preferred_element_type=f32)
        o_ref[:, D:] = jnp.dot(av, w2_ref[...], preferred_element_type=f32)

    return pl.pallas_call(
        body,
        grid=(n // bn,),
        in_specs=[
            pl.BlockSpec((bn, D), lambda i: (i, 0)),
            pl.BlockSpec((D, D), lambda i: (0, 0)),
            pl.BlockSpec((D, D), lambda i: (0, 0)),
        ],
        out_specs=pl.BlockSpec((bn, 2 * D), lambda i: (i, 0)),
        out_shape=jax.ShapeDtypeStruct((n, 2 * D), f32),
    )(a, w1, w2)


def _mm2(a, w1, w2):
    n = a.shape[0]
    bn = 200
    assert n % bn == 0

    def body(a_ref, w1_ref, w2_ref, o1_ref, o2_ref):
        av = a_ref[...]
        o1_ref[...] = jnp.dot(av, w1_ref[...], preferred_element_type=f32)
        o2_ref[...] = jnp.dot(av, w2_ref[...], preferred_element_type=f32)

    return pl.pallas_call(
        body,
        grid=(n // bn,),
        in_specs=[
            pl.BlockSpec((bn, D), lambda i: (i, 0)),
            pl.BlockSpec((D, D), lambda i: (0, 0)),
            pl.BlockSpec((D, D), lambda i: (0, 0)),
        ],
        out_specs=[
            pl.BlockSpec((bn, D), lambda i: (i, 0)),
            pl.BlockSpec((bn, D), lambda i: (i, 0)),
        ],
        out_shape=[
            jax.ShapeDtypeStruct((n, D), f32),
            jax.ShapeDtypeStruct((n, D), f32),
        ],
    )(a, w1, w2)


def _edge_math(kve, qe, t3, ea_table, et_table, wea, wet, e_real):
    ep = kve.shape[0]
    be = 512
    assert ep % be == 0
    scale = 1.0 / math.sqrt(DH)

    def body(kve_ref, qe_ref, t_ref, ea_ref, et_ref, wea_ref, wet_ref,
             wv_ref, ww_ref):
        pid = pl.program_id(0)
        tk = jnp.dot(ea_ref[...], wea_ref[...], preferred_element_type=f32)
        tv = jnp.dot(et_ref[...], wet_ref[...], preferred_element_type=f32)
        t = t_ref[0, 0, :]
        oh = (t[:, None] == lax.broadcasted_iota(i32, (be, 4), 1)).astype(f32)
        ke = kve_ref[:, :D] + jnp.dot(oh, tk, preferred_element_type=f32)
        ve = kve_ref[:, D:] + jnp.dot(oh, tv, preferred_element_type=f32)
        hsel = (lax.broadcasted_iota(i32, (D, H), 0) // DH
                == lax.broadcasted_iota(i32, (D, H), 1)).astype(f32)
        logits = jnp.dot(qe_ref[...] * ke, hsel,
                         preferred_element_type=f32) * scale
        w = jnp.exp(logits)
        eid = pid * be + lax.broadcasted_iota(i32, (be, 1), 0)
        w = jnp.where(eid < e_real, w, 0.0)
        wexp = jnp.dot(w, hsel.T, preferred_element_type=f32)
        wv_ref[...] = wexp * ve
        ww_ref[...] = wexp

    return pl.pallas_call(
        body,
        grid=(ep // be,),
        in_specs=[
            pl.BlockSpec((be, 2 * D), lambda i: (i, 0)),
            pl.BlockSpec((be, D), lambda i: (i, 0)),
            pl.BlockSpec((1, 1, be), lambda i: (i, 0, 0)),
            pl.BlockSpec((4, D), lambda i: (0, 0)),
            pl.BlockSpec((4, D), lambda i: (0, 0)),
            pl.BlockSpec((D, D), lambda i: (0, 0)),
            pl.BlockSpec((D, D), lambda i: (0, 0)),
        ],
        out_specs=[
            pl.BlockSpec((be, D), lambda i: (i, 0)),
            pl.BlockSpec((be, D), lambda i: (i, 0)),
        ],
        out_shape=[
            jax.ShapeDtypeStruct((ep, D), f32),
            jax.ShapeDtypeStruct((ep, D), f32),
        ],
    )(kve, qe, t3, ea_table, et_table, wea, wet)


def _epilogue(accs, wo, skip, g, b, n_dst, n_dst_p, with_bn):
    accv, accw = accs

    def body(accv_ref, accw_ref, wo_ref, skip_ref, g_ref, b_ref, o_ref):
        agg = accv_ref[pl.ds(0, n_dst), :] + accv_ref[pl.ds(n_dst_p, n_dst), :]
        den = (accw_ref[pl.ds(0, n_dst), :]
               + accw_ref[pl.ds(n_dst_p, n_dst), :] + 1e-16)
        h = jnp.dot(agg / den, wo_ref[...], preferred_element_type=f32)
        h = h + skip_ref[...]
        if with_bn:
            mu = jnp.mean(h, axis=0, keepdims=True)
            var = jnp.mean((h - mu) * (h - mu), axis=0, keepdims=True)
            h = (h - mu) * lax.rsqrt(var + 1e-5) * g_ref[...] + b_ref[...]
        o_ref[...] = h

    return pl.pallas_call(
        body,
        out_shape=jax.ShapeDtypeStruct((n_dst, D), f32),
    )(accv, accw, wo, skip, g, b)


@functools.cache
def _make_gather(ep, n_chunks):
    mesh = plsc.VectorSubcoreMesh(core_axis_name="c", subcore_axis_name="s")
    assert n_chunks % 2 == 0

    @functools.partial(
        pl.kernel,
        out_type=[
            jax.ShapeDtypeStruct((ep, 2 * D), f32),
            jax.ShapeDtypeStruct((ep, D), f32),
        ],
        mesh=mesh,
        scratch_types=[
            pltpu.VMEM((n_chunks * CH,), i32),
            pltpu.VMEM((n_chunks * CH,), i32),
            pltpu.VMEM((CH, 2 * D), f32),
            pltpu.VMEM((CH, 2 * D), f32),
            pltpu.VMEM((CH, D), f32),
            pltpu.VMEM((CH, D), f32),
            pltpu.SemaphoreType.DMA,
            pltpu.SemaphoreType.DMA,
            pltpu.SemaphoreType.DMA,
            pltpu.SemaphoreType.DMA,
        ],
    )
    def k(src1d, dst1d, kv_hbm, q_hbm, kve_out, qe_out,
          sidx, didx, kva, kvb, qa, qb, semkva, semkvb, semqa, semqb):
        wid = lax.axis_index("s") * NC + lax.axis_index("c")
        rbase = wid * n_chunks
        pltpu.sync_copy(src1d.at[pl.ds(rbase * CH, n_chunks * CH)], sidx)
        pltpu.sync_copy(dst1d.at[pl.ds(rbase * CH, n_chunks * CH)], didx)

        def fire(j, kvbuf, qbuf, semkv, semq):
            pltpu.async_copy(kv_hbm.at[sidx.at[pl.ds(j * CH, CH)]], kvbuf, semkv)
            pltpu.async_copy(q_hbm.at[didx.at[pl.ds(j * CH, CH)]], qbuf, semq)

        def drain(kvbuf, qbuf, semkv, semq):
            pltpu.make_async_copy(kv_hbm.at[pl.ds(0, CH)], kvbuf, semkv).wait()
            pltpu.make_async_copy(q_hbm.at[pl.ds(0, CH)], qbuf, semq).wait()

        def put(j, kvbuf, qbuf):
            ebase = (rbase + j) * CH
            pltpu.sync_copy(kvbuf, kve_out.at[pl.ds(ebase, CH)])
            pltpu.sync_copy(qbuf, qe_out.at[pl.ds(ebase, CH)])

        fire(0, kva, qa, semkva, semqa)

        def body(jj, carry):
            j0 = 2 * jj
            j1 = j0 + 1
            j2 = jnp.minimum(j0 + 2, n_chunks - 1)
            fire(j1, kvb, qb, semkvb, semqb)
            drain(kva, qa, semkva, semqa)
            put(j0, kva, qa)
            fire(j2, kva, qa, semkva, semqa)
            drain(kvb, qb, semkvb, semqb)
            put(j1, kvb, qb)
            return carry

        lax.fori_loop(0, n_chunks // 2, body, 0)
        drain(kva, qa, semkva, semqa)

    return k


@functools.cache
def _make_scatter(ep, n_chunks, n_dst_p):
    mesh = plsc.VectorSubcoreMesh(core_axis_name="c", subcore_axis_name="s")
    rpt = n_dst_p // NS
    dr = 64
    assert rpt % dr == 0

    @functools.partial(
        pl.kernel,
        out_type=jax.ShapeDtypeStruct((NC * n_dst_p, D), f32),
        mesh=mesh,
        scratch_types=[
            pltpu.VMEM((CH,), i32),
            pltpu.VMEM((CH,), i32),
            pltpu.VMEM((CH, D), f32),
            pltpu.VMEM((CH, D), f32),
            pltpu.VMEM((dr, D), f32),
            pltpu.VMEM_SHARED((n_dst_p, D), f32),
            pltpu.SemaphoreType.DMA,
            pltpu.SemaphoreType.DMA,
            pltpu.SemaphoreType.DMA,
            pltpu.SemaphoreType.DMA,
        ],
    )
    def k(dst1d, wv_hbm, acc_out, drowa, drowb, bufa, bufb, zbuf, shared,
          semba, sembb, semia, semib):
        cid = lax.axis_index("c")
        sid = lax.axis_index("s")
        wid = sid * NC + cid
        rbase = wid * n_chunks

        def zbody(r, carry):
            for c in range(D // 16):
                zbuf[r, pl.ds(c * 16, 16)] = jnp.zeros((16,), f32)
            return carry

        lax.fori_loop(0, dr, zbody, 0)
        for z in range(rpt // dr):
            pltpu.sync_copy(zbuf, shared.at[pl.ds(sid * rpt + z * dr, dr)])
        plsc.subcore_barrier()

        def fire(j, buf, drow, semb, semi):
            ebase = (rbase + j) * CH
            pltpu.async_copy(wv_hbm.at[pl.ds(ebase, CH)], buf, semb)
            pltpu.async_copy(dst1d.at[pl.ds(ebase, CH)], drow, semi)

        def drain(buf, drow, semb, semi):
            pltpu.make_async_copy(wv_hbm.at[pl.ds(0, CH)], buf, semb).wait()
            pltpu.make_async_copy(dst1d.at[pl.ds(0, CH)], drow, semi).wait()

        fire(0, bufa, drowa, semba, semia)

        def body(jj, carry):
            j0 = 2 * jj
            j1 = j0 + 1
            j2 = jnp.minimum(j0 + 2, n_chunks - 1)
            fire(j1, bufb, drowb, sembb, semib)
            drain(bufa, drowa, semba, semia)
            pltpu.sync_copy(bufa, shared.at[drowa], add=True)
            fire(j2, bufa, drowa, semba, semia)
            drain(bufb, drowb, sembb, semib)
            pltpu.sync_copy(bufb, shared.at[drowb], add=True)
            return carry

        lax.fori_loop(0, n_chunks // 2, body, 0)
        drain(bufa, drowa, semba, semia)
        plsc.subcore_barrier()

        for z in range(rpt // dr):
            pltpu.sync_copy(shared.at[pl.ds(sid * rpt + z * dr, dr)], zbuf)
            pltpu.sync_copy(
                zbuf, acc_out.at[pl.ds(cid * n_dst_p + sid * rpt + z * dr, dr)])

    return k


def _pad_edges(src, dst, typ, ep):
    e = src.shape[0]
    pad = ep - e
    src = jnp.concatenate([src, jnp.zeros((pad,), i32)])
    dst = jnp.concatenate([dst, jnp.zeros((pad,), i32)])
    typ = jnp.concatenate([typ, jnp.zeros((pad,), i32)])
    return src, dst, typ.reshape(ep // 512, 1, 512)


def _layer(x_src, x_dst, src2d, dst2d, t3, e_real, ep, n_dst, n_dst_p,
           ea_table, et_table, wq, wk, wv, wea, wet, wo, ws, g, b, with_bn):
    n_chunks = ep // (NW * CH)
    kv = _mmcat(x_src, wk, wv)
    q, skip = _mm2(x_dst, wq, ws)
    kve, qe = _make_gather(ep, n_chunks)(src2d, dst2d, kv, q)
    wvz, wwz = _edge_math(kve, qe, t3, ea_table, et_table, wea, wet, e_real)
    scat = _make_scatter(ep, n_chunks, n_dst_p)
    accv = scat(dst2d, wvz)
    accw = scat(dst2d, wwz)
    return _epilogue((accv, accw), wo, skip, g, b, n_dst, n_dst_p, with_bn)


def kernel(x, edge_index0, edge_type0, edge_index1, edge_type1,
           et_table, ea_table, g0, b0,
           Wq0, Wk0, Wv0, Wea0, Wet0, Wo0, Ws0,
           Wq1, Wk1, Wv1, Wea1, Wet1, Wo1, Ws1):
    ep0 = _ceil_to(E0, NW * CH * 4)
    ep1 = _ceil_to(E1, NW * CH * 4)
    n1p = _ceil_to(N1, NS * 8)
    n2p = _ceil_to(N2, NS * 8)
    g2 = g0.reshape(1, D)
    b2 = b0.reshape(1, D)

    s0, d0, t0 = _pad_edges(edge_index0[0], edge_index0[1], edge_type0, ep0)
    s1, d1, t1 = _pad_edges(edge_index1[0], edge_index1[1], edge_type1, ep1)

    h1 = _layer(x, x[:N1], s0, d0, t0, E0, ep0, N1, n1p,
                ea_table, et_table, Wq0, Wk0, Wv0, Wea0, Wet0, Wo0, Ws0,
                g2, b2, True)
    out = _layer(h1, h1[:N2], s1, d1, t1, E1, ep1, N2, n2p,
                 ea_table, et_table, Wq1, Wk1, Wv1, Wea1, Wet1, Wo1, Ws1,
                 g2, b2, False)
    return out
